# speedup vs baseline: 8.9623x; 8.9623x over previous
import functools

import jax
import jax.numpy as jnp
from jax import lax
from jax.experimental import pallas as pl
from jax.experimental.pallas import tpu as pltpu
from jax.experimental.pallas import tpu_sc as plsc

N = 10000
D = 128
E = 320000
NUM_CONVS = 6
NUM_BN = 5

NC = 2
NS = 16
NW = NC * NS
C = 128
K = -(-E // (NW * C))
E_PAD = NW * K * C
ROWS_PER_TILE = 640
N_PAD = NS * ROWS_PER_TILE

_mesh = plsc.VectorSubcoreMesh(core_axis_name="c", subcore_axis_name="s")


@functools.partial(
    pl.kernel,
    out_type=jax.ShapeDtypeStruct((NC, N_PAD, 16), jnp.float32),
    mesh=_mesh,
    scratch_types=[
        pltpu.VMEM((K, C), jnp.int32),
        pltpu.VMEM((C, 16), jnp.float32),
        pltpu.VMEM_SHARED((N_PAD, 16), jnp.float32),
    ],
)
def _sc_degree(dstb_hbm, zeros_hbm, ones_hbm, out_hbm, didx, ones_v, acc):
    c = lax.axis_index("c")
    s = lax.axis_index("s")
    wid = s * NC + c
    base = s * ROWS_PER_TILE
    pltpu.sync_copy(dstb_hbm.at[wid], didx)
    pltpu.sync_copy(ones_hbm, ones_v)
    pltpu.sync_copy(zeros_hbm, acc.at[pl.ds(base, ROWS_PER_TILE)])
    plsc.subcore_barrier()

    def body(k, carry):
        pltpu.sync_copy(ones_v, acc.at[didx.at[k]], add=True)
        return carry

    lax.fori_loop(0, K, body, 0)
    plsc.subcore_barrier()
    pltpu.sync_copy(acc.at[pl.ds(base, ROWS_PER_TILE)],
                    out_hbm.at[c, pl.ds(base, ROWS_PER_TILE)])


@functools.partial(
    pl.kernel,
    out_type=jax.ShapeDtypeStruct((NC, N_PAD, D), jnp.float32),
    mesh=_mesh,
    scratch_types=[
        pltpu.VMEM((K, C), jnp.int32),
        pltpu.VMEM((K, C), jnp.int32),
        pltpu.VMEM((C, D), jnp.float32),
        pltpu.VMEM_SHARED((N_PAD, D), jnp.float32),
        pltpu.SemaphoreType.DMA,
    ],
)
def _sc_aggregate(g_hbm, srcb_hbm, dstb_hbm, zeros_hbm, out_hbm,
                  sidx, didx, rows, acc, sem):
    c = lax.axis_index("c")
    s = lax.axis_index("s")
    wid = s * NC + c
    base = s * ROWS_PER_TILE
    pltpu.sync_copy(srcb_hbm.at[wid], sidx)
    pltpu.sync_copy(dstb_hbm.at[wid], didx)
    pltpu.sync_copy(zeros_hbm, acc.at[pl.ds(base, ROWS_PER_TILE)])
    plsc.subcore_barrier()

    def body(k, carry):
        pltpu.async_copy(g_hbm.at[sidx.at[k]], rows, sem).wait()
        pltpu.sync_copy(rows, acc.at[didx.at[k]], add=True)
        return carry

    lax.fori_loop(0, K, body, 0)
    plsc.subcore_barrier()
    pltpu.sync_copy(acc.at[pl.ds(base, ROWS_PER_TILE)],
                    out_hbm.at[c, pl.ds(base, ROWS_PER_TILE)])


def _tc_first_body(x_ref, dpart_ref, w_ref, dinv_ref, g_ref):
    deg = dpart_ref[0] + dpart_ref[1] + 1.0
    dinv = lax.rsqrt(deg)
    dinv_ref[...] = dinv
    h1 = jnp.dot(x_ref[...], w_ref[...], preferred_element_type=jnp.float32)
    g_ref[...] = dinv[:N, 0:1] * h1


def _tc_mid_body(part_ref, g_ref, dinv_ref, w_ref, b_ref, gam_ref, bet_ref,
                 gout_ref):
    dinv = dinv_ref[:N, 0:1]
    sagg = part_ref[0, :N, :] + part_ref[1, :N, :] + g_ref[...]
    h = dinv * sagg + b_ref[...]
    mean = jnp.mean(h, axis=0, keepdims=True)
    var = jnp.mean(h * h, axis=0, keepdims=True) - mean * mean
    h = (h - mean) * lax.rsqrt(var + 1e-5) * gam_ref[...] + bet_ref[...]
    h = jnp.maximum(h, 0.0)
    h1 = jnp.dot(h, w_ref[...], preferred_element_type=jnp.float32)
    gout_ref[...] = dinv * h1


def _tc_last_body(part_ref, g_ref, dinv_ref, b_ref, out_ref):
    dinv = dinv_ref[:N, 0:1]
    sagg = part_ref[0, :N, :] + part_ref[1, :N, :] + g_ref[...]
    out_ref[...] = dinv * sagg + b_ref[...]


_tc_first = pl.pallas_call(
    _tc_first_body,
    out_shape=(jax.ShapeDtypeStruct((N_PAD, 16), jnp.float32),
               jax.ShapeDtypeStruct((N, D), jnp.float32)),
)

_tc_mid = pl.pallas_call(
    _tc_mid_body,
    out_shape=jax.ShapeDtypeStruct((N, D), jnp.float32),
)

_tc_last = pl.pallas_call(
    _tc_last_body,
    out_shape=jax.ShapeDtypeStruct((N, D), jnp.float32),
)


def kernel(x, edge_index, Ws, bs, gammas, betas):
    src = edge_index[0]
    dst = edge_index[1]
    pad = E_PAD - E
    srcb = jnp.concatenate([src, jnp.zeros((pad,), jnp.int32)]).reshape(NW, K, C)
    dstb = jnp.concatenate([dst, jnp.full((pad,), N, jnp.int32)]).reshape(NW, K, C)

    zeros16 = jnp.zeros((ROWS_PER_TILE, 16), jnp.float32)
    ones16 = jnp.ones((C, 16), jnp.float32)
    zerosD = jnp.zeros((ROWS_PER_TILE, D), jnp.float32)

    dpart = _sc_degree(dstb, zeros16, ones16)
    dinv, g = _tc_first(x, dpart, Ws[0])

    for i in range(NUM_BN):
        part = _sc_aggregate(g, srcb, dstb, zerosD)
        g = _tc_mid(part, g, dinv, Ws[i + 1],
                    bs[i].reshape(1, D), gammas[i].reshape(1, D),
                    betas[i].reshape(1, D))

    part = _sc_aggregate(g, srcb, dstb, zerosD)
    return _tc_last(part, g, dinv, bs[NUM_CONVS - 1].reshape(1, D))
